# baseline (speedup 1.0000x reference)
import functools

import jax
import jax.numpy as jnp
from jax import lax
from jax.experimental import pallas as pl
from jax.experimental.pallas import tpu as pltpu
import jax.experimental.pallas.tpu_sc as plsc

_NC_SC = 2
_NS = 16
_CHUNK = 80


def _fill_const(ref, nrows, ncols, value):
    groups = ncols // 16

    def fill(i, _):
        for j in range(groups):
            ref[i, pl.ds(j * 16, 16)] = jnp.full((16,), value, jnp.float32)
        return 0
    lax.fori_loop(0, nrows, fill, 0)


def _spmem_rows(N):
    rpt = (N // _NS) // 8 * 8
    tail0 = rpt * _NS
    return rpt, tail0, N - tail0


def _zero_spmem(acc, stg, s, rpt, tail0, tailn):
    r0 = s * rpt
    nfull, rem = rpt // _CHUNK, rpt % _CHUNK

    def z(k, _):
        pltpu.sync_copy(stg, acc.at[pl.ds(r0 + k * _CHUNK, _CHUNK)])
        return 0
    lax.fori_loop(0, nfull, z, 0)
    if rem:
        pltpu.sync_copy(stg.at[pl.ds(0, rem)], acc.at[pl.ds(r0 + nfull * _CHUNK, rem)])

    @pl.when(s == 0)
    def _():
        pltpu.sync_copy(stg.at[pl.ds(0, tailn)], acc.at[pl.ds(tail0, tailn)])


def _drain_spmem(acc, stg, out_slice, s, rpt, tail0, tailn):
    r0 = s * rpt
    nfull, rem = rpt // _CHUNK, rpt % _CHUNK

    def d(k, _):
        off = r0 + k * _CHUNK
        pltpu.sync_copy(acc.at[pl.ds(off, _CHUNK)], stg)
        pltpu.sync_copy(stg, out_slice.at[pl.ds(off, _CHUNK)])
        return 0
    lax.fori_loop(0, nfull, d, 0)
    if rem:
        off = r0 + nfull * _CHUNK
        pltpu.sync_copy(acc.at[pl.ds(off, rem)], stg.at[pl.ds(0, rem)])
        pltpu.sync_copy(stg.at[pl.ds(0, rem)], out_slice.at[pl.ds(off, rem)])

    @pl.when(s == 0)
    def _():
        pltpu.sync_copy(acc.at[pl.ds(tail0, tailn)], stg.at[pl.ds(0, tailn)])
        pltpu.sync_copy(stg.at[pl.ds(0, tailn)], out_slice.at[pl.ds(tail0, tailn)])


def _make_agg1(N, D, E):
    EPC = E // _NC_SC
    EPT = EPC // _NS
    NCH = EPT // _CHUNK
    RPT, TAIL0, TAILN = _spmem_rows(N)
    mesh = plsc.VectorSubcoreMesh(
        core_axis_name="c", subcore_axis_name="s",
        num_cores=_NC_SC, num_subcores=_NS)

    @functools.partial(
        pl.kernel,
        out_type=(jax.ShapeDtypeStruct((_NC_SC, N, D), jnp.float32),
                  jax.ShapeDtypeStruct((_NC_SC, N, D), jnp.float32)),
        mesh=mesh,
        scratch_types=[
            pltpu.VMEM((_CHUNK,), jnp.int32),
            pltpu.VMEM((_CHUNK,), jnp.int32),
            pltpu.VMEM((_CHUNK, D), jnp.float32),
            pltpu.VMEM((_CHUNK, D), jnp.float32),
            pltpu.VMEM_SHARED((N, D), jnp.float32),
            pltpu.SemaphoreType.DMA,
        ],
    )
    def k(x_hbm, src_hbm, dst_hbm, sums_out, cnts_out,
          isrc, idst, rows, stg, acc, sem):
        c = lax.axis_index("c")
        s = lax.axis_index("s")

        _fill_const(stg, _CHUNK, D, 0.0)
        _zero_spmem(acc, stg, s, RPT, TAIL0, TAILN)
        _fill_const(stg, _CHUNK, D, 1.0)
        plsc.subcore_barrier()

        base = c * EPC + s * EPT

        def cbody(i, _):
            off = base + i * _CHUNK
            pltpu.sync_copy(dst_hbm.at[pl.ds(off, _CHUNK)], idst)
            pltpu.sync_copy(stg, acc.at[idst], add=True)
            return 0
        lax.fori_loop(0, NCH, cbody, 0)

        plsc.subcore_barrier()
        _drain_spmem(acc, rows, cnts_out.at[c], s, RPT, TAIL0, TAILN)
        _fill_const(stg, _CHUNK, D, 0.0)
        _zero_spmem(acc, stg, s, RPT, TAIL0, TAILN)
        plsc.subcore_barrier()

        def body(i, _):
            off = base + i * _CHUNK
            pltpu.sync_copy(src_hbm.at[pl.ds(off, _CHUNK)], isrc)
            pltpu.sync_copy(dst_hbm.at[pl.ds(off, _CHUNK)], idst)
            pltpu.async_copy(x_hbm.at[isrc], rows, sem).wait()
            pltpu.sync_copy(rows, acc.at[idst], add=True)
            return 0
        lax.fori_loop(0, NCH, body, 0)

        plsc.subcore_barrier()
        _drain_spmem(acc, rows, sums_out.at[c], s, RPT, TAIL0, TAILN)

    return k


def _make_agg2(N, Dh, E):
    EPT = E // _NS
    NCH = EPT // _CHUNK
    RPT, TAIL0, TAILN = _spmem_rows(N)
    mesh = plsc.VectorSubcoreMesh(
        core_axis_name="c", subcore_axis_name="s",
        num_cores=_NC_SC, num_subcores=_NS)

    @functools.partial(
        pl.kernel,
        out_type=jax.ShapeDtypeStruct((_NC_SC, N, Dh), jnp.float32),
        mesh=mesh,
        scratch_types=[
            pltpu.VMEM((_CHUNK,), jnp.int32),
            pltpu.VMEM((_CHUNK,), jnp.int32),
            pltpu.VMEM((_CHUNK, Dh), jnp.float32),
            pltpu.VMEM((_CHUNK, Dh), jnp.float32),
            pltpu.VMEM_SHARED((N, Dh), jnp.float32),
            pltpu.SemaphoreType.DMA,
        ],
    )
    def k(tab_hbm, src_hbm, dst_hbm, sums_out,
          isrc, idst, rows, stg, acc, sem):
        c = lax.axis_index("c")
        s = lax.axis_index("s")

        _fill_const(stg, _CHUNK, Dh, 0.0)
        _zero_spmem(acc, stg, s, RPT, TAIL0, TAILN)
        plsc.subcore_barrier()

        base = s * EPT
        shift = c * N

        def body(i, _):
            off = base + i * _CHUNK
            pltpu.sync_copy(src_hbm.at[pl.ds(off, _CHUNK)], isrc)
            pltpu.sync_copy(dst_hbm.at[pl.ds(off, _CHUNK)], idst)
            for j in range(_CHUNK // 16):
                sl = pl.ds(j * 16, 16)
                isrc[sl] = isrc[sl] + shift
            pltpu.async_copy(tab_hbm.at[isrc], rows, sem).wait()
            pltpu.sync_copy(rows, acc.at[idst], add=True)
            return 0
        lax.fori_loop(0, NCH, body, 0)

        plsc.subcore_barrier()
        _drain_spmem(acc, stg, sums_out.at[c], s, RPT, TAIL0, TAILN)

    return k


def _stageB_body(sums_ref, cnts_ref, x_ref, w1l_ref, b1_ref, w1r_ref,
                 w2l_ref, w2r_ref, p2_ref, r2_ref, *, half):
    ssum = sums_ref[0] + sums_ref[1]
    cnt = cnts_ref[0, :, 0:1] + cnts_ref[1, :, 0:1]
    inv = 1.0 / jnp.maximum(cnt, 1.0)
    mean = ssum * inv
    h1 = jnp.maximum(
        jnp.dot(mean, w1l_ref[...], preferred_element_type=jnp.float32)
        + jnp.dot(x_ref[...], w1r_ref[...], preferred_element_type=jnp.float32)
        + b1_ref[...], 0.0)
    p2 = jnp.dot(h1, w2l_ref[...], preferred_element_type=jnp.float32)
    p2_ref[0] = p2[:, :half]
    p2_ref[1] = p2[:, half:]
    r2_ref[...] = jnp.dot(h1, w2r_ref[...], preferred_element_type=jnp.float32)


def _stageC_body(sums2_ref, cnts_ref, r2_ref, b2_ref, wh0_ref, bh0_ref,
                 wh1_ref, bh1_ref, wh2_ref, bh2_ref, wf_ref, bf_ref,
                 out_ref, *, n_cls):
    cnt = cnts_ref[0, :, 0:1] + cnts_ref[1, :, 0:1]
    inv = 1.0 / jnp.maximum(cnt, 1.0)
    mean2 = jnp.concatenate([sums2_ref[0], sums2_ref[1]], axis=1) * inv
    h = jnp.maximum(mean2 + b2_ref[...] + r2_ref[...], 0.0)
    h = jnp.maximum(jnp.dot(h, wh0_ref[...], preferred_element_type=jnp.float32) + bh0_ref[...], 0.0)
    h = jnp.maximum(jnp.dot(h, wh1_ref[...], preferred_element_type=jnp.float32) + bh1_ref[...], 0.0)
    h = jnp.maximum(jnp.dot(h, wh2_ref[...], preferred_element_type=jnp.float32) + bh2_ref[...], 0.0)
    logits = jnp.dot(h, wf_ref[...], preferred_element_type=jnp.float32) + bf_ref[...]
    col = lax.broadcasted_iota(jnp.int32, logits.shape, 1)
    masked = jnp.where(col < n_cls, logits, -1e30)
    m = jnp.max(masked, axis=1, keepdims=True)
    e = jnp.exp(masked - m)
    probs = e / jnp.sum(e, axis=1, keepdims=True)
    out_ref[...] = probs[:, :n_cls]


def kernel(x, edge_index, W1l, b1, W1r, W2l, b2, W2r,
           Wh0, bh0, Wh1, bh1, Wh2, bh2, Wf, bf):
    N, D = x.shape
    E = edge_index.shape[1]
    H1 = W1l.shape[1]
    H2 = W2l.shape[1]
    NCLS = Wf.shape[1]
    half = H2 // 2
    BN = 400
    grid = (N // BN,)

    src = edge_index[0].astype(jnp.int32)
    dst = edge_index[1].astype(jnp.int32)

    sums1, cnts1 = _make_agg1(N, D, E)(x, src, dst)

    wcopy = lambda shape: pl.BlockSpec(shape, lambda i: tuple(0 for _ in shape))
    b1r = b1.reshape(1, H1)
    stageB = pl.pallas_call(
        functools.partial(_stageB_body, half=half),
        grid=grid,
        in_specs=[
            pl.BlockSpec((_NC_SC, BN, D), lambda i: (0, i, 0)),
            pl.BlockSpec((_NC_SC, BN, D), lambda i: (0, i, 0)),
            pl.BlockSpec((BN, D), lambda i: (i, 0)),
            wcopy((D, H1)), wcopy((1, H1)), wcopy((D, H1)),
            wcopy((H1, H2)), wcopy((H1, H2)),
        ],
        out_specs=[
            pl.BlockSpec((_NC_SC, BN, half), lambda i: (0, i, 0)),
            pl.BlockSpec((BN, H2), lambda i: (i, 0)),
        ],
        out_shape=[
            jax.ShapeDtypeStruct((_NC_SC, N, half), jnp.float32),
            jax.ShapeDtypeStruct((N, H2), jnp.float32),
        ],
    )
    p2, r2 = stageB(sums1, cnts1, x, W1l, b1r, W1r, W2l, W2r)

    sums2 = _make_agg2(N, half, E)(p2.reshape(_NC_SC * N, half), src, dst)

    wf_p = jnp.zeros((H2, 128), jnp.float32).at[:, :NCLS].set(Wf)
    bf_p = jnp.zeros((1, 128), jnp.float32).at[0, :NCLS].set(bf)
    stageC = pl.pallas_call(
        functools.partial(_stageC_body, n_cls=NCLS),
        grid=grid,
        in_specs=[
            pl.BlockSpec((_NC_SC, BN, half), lambda i: (0, i, 0)),
            pl.BlockSpec((_NC_SC, BN, 128), lambda i: (0, i, 0)),
            pl.BlockSpec((BN, H2), lambda i: (i, 0)),
            wcopy((1, H2)),
            wcopy((H2, H2)), wcopy((1, H2)),
            wcopy((H2, H2)), wcopy((1, H2)),
            wcopy((H2, H2)), wcopy((1, H2)),
            wcopy((H2, 128)), wcopy((1, 128)),
        ],
        out_specs=pl.BlockSpec((BN, NCLS), lambda i: (i, 0)),
        out_shape=jax.ShapeDtypeStruct((N, NCLS), jnp.float32),
    )
    return stageC(sums2, cnts1, r2, b2.reshape(1, H2),
                  Wh0, bh0.reshape(1, H2), Wh1, bh1.reshape(1, H2),
                  Wh2, bh2.reshape(1, H2), wf_p, bf_p)


# speedup vs baseline: 1.3124x; 1.3124x over previous
import functools

import jax
import jax.numpy as jnp
from jax import lax
from jax.experimental import pallas as pl
from jax.experimental.pallas import tpu as pltpu
import jax.experimental.pallas.tpu_sc as plsc

_NC_SC = 2
_NS = 16
_CHUNK = 80


def _fill_const(ref, nrows, ncols, value):
    groups = ncols // 16

    def fill(i, _):
        for j in range(groups):
            ref[i, pl.ds(j * 16, 16)] = jnp.full((16,), value, jnp.float32)
        return 0
    lax.fori_loop(0, nrows, fill, 0)


def _spmem_rows(N):
    rpt = (N // _NS) // 8 * 8
    tail0 = rpt * _NS
    return rpt, tail0, N - tail0


def _zero_spmem(acc, stg, s, rpt, tail0, tailn):
    r0 = s * rpt
    nfull, rem = rpt // _CHUNK, rpt % _CHUNK

    def z(k, _):
        pltpu.sync_copy(stg, acc.at[pl.ds(r0 + k * _CHUNK, _CHUNK)])
        return 0
    lax.fori_loop(0, nfull, z, 0)
    if rem:
        pltpu.sync_copy(stg.at[pl.ds(0, rem)], acc.at[pl.ds(r0 + nfull * _CHUNK, rem)])

    @pl.when(s == 0)
    def _():
        pltpu.sync_copy(stg.at[pl.ds(0, tailn)], acc.at[pl.ds(tail0, tailn)])


def _drain_spmem(acc, stg, out_slice, s, rpt, tail0, tailn):
    r0 = s * rpt
    nfull, rem = rpt // _CHUNK, rpt % _CHUNK

    def d(k, _):
        off = r0 + k * _CHUNK
        pltpu.sync_copy(acc.at[pl.ds(off, _CHUNK)], stg)
        pltpu.sync_copy(stg, out_slice.at[pl.ds(off, _CHUNK)])
        return 0
    lax.fori_loop(0, nfull, d, 0)
    if rem:
        off = r0 + nfull * _CHUNK
        pltpu.sync_copy(acc.at[pl.ds(off, rem)], stg.at[pl.ds(0, rem)])
        pltpu.sync_copy(stg.at[pl.ds(0, rem)], out_slice.at[pl.ds(off, rem)])

    @pl.when(s == 0)
    def _():
        pltpu.sync_copy(acc.at[pl.ds(tail0, tailn)], stg.at[pl.ds(0, tailn)])
        pltpu.sync_copy(stg.at[pl.ds(0, tailn)], out_slice.at[pl.ds(tail0, tailn)])


def _make_agg1(N, D, E):
    EPC = E // _NC_SC
    EPT = EPC // _NS
    NCH = EPT // _CHUNK
    RPT, TAIL0, TAILN = _spmem_rows(N)
    mesh = plsc.VectorSubcoreMesh(
        core_axis_name="c", subcore_axis_name="s",
        num_cores=_NC_SC, num_subcores=_NS)

    @functools.partial(
        pl.kernel,
        out_type=(jax.ShapeDtypeStruct((_NC_SC, N, D), jnp.float32),
                  jax.ShapeDtypeStruct((_NC_SC, N, D), jnp.float32)),
        mesh=mesh,
        scratch_types=[
            pltpu.VMEM((_CHUNK,), jnp.int32),
            pltpu.VMEM((_CHUNK,), jnp.int32),
            pltpu.VMEM((_CHUNK,), jnp.int32),
            pltpu.VMEM((_CHUNK,), jnp.int32),
            pltpu.VMEM((_CHUNK, D), jnp.float32),
            pltpu.VMEM((_CHUNK, D), jnp.float32),
            pltpu.VMEM((_CHUNK, D), jnp.float32),
            pltpu.VMEM_SHARED((N, D), jnp.float32),
            pltpu.SemaphoreType.DMA,
            pltpu.SemaphoreType.DMA,
        ],
    )
    def k(x_hbm, src_hbm, dst_hbm, sums_out, cnts_out,
          isrc, idst, isrc2, idst2, rows, rows2, stg, acc, sem, sem2):
        c = lax.axis_index("c")
        s = lax.axis_index("s")

        _fill_const(stg, _CHUNK, D, 0.0)
        _zero_spmem(acc, stg, s, RPT, TAIL0, TAILN)
        _fill_const(stg, _CHUNK, D, 1.0)
        plsc.subcore_barrier()

        base = c * EPC + s * EPT

        def cbody(i, _):
            off = base + i * _CHUNK
            pltpu.sync_copy(dst_hbm.at[pl.ds(off, _CHUNK)], idst)
            pltpu.sync_copy(stg, acc.at[idst], add=True)
            return 0
        lax.fori_loop(0, NCH, cbody, 0)

        plsc.subcore_barrier()
        _drain_spmem(acc, rows, cnts_out.at[c], s, RPT, TAIL0, TAILN)
        _fill_const(stg, _CHUNK, D, 0.0)
        _zero_spmem(acc, stg, s, RPT, TAIL0, TAILN)
        plsc.subcore_barrier()

        def body(k2, _):
            off0 = base + (2 * k2) * _CHUNK
            off1 = off0 + _CHUNK
            pltpu.sync_copy(src_hbm.at[pl.ds(off0, _CHUNK)], isrc)
            pltpu.sync_copy(dst_hbm.at[pl.ds(off0, _CHUNK)], idst)
            cp0 = pltpu.async_copy(x_hbm.at[isrc], rows, sem)
            pltpu.sync_copy(src_hbm.at[pl.ds(off1, _CHUNK)], isrc2)
            pltpu.sync_copy(dst_hbm.at[pl.ds(off1, _CHUNK)], idst2)
            cp1 = pltpu.async_copy(x_hbm.at[isrc2], rows2, sem2)
            cp0.wait()
            pltpu.sync_copy(rows, acc.at[idst], add=True)
            cp1.wait()
            pltpu.sync_copy(rows2, acc.at[idst2], add=True)
            return 0
        lax.fori_loop(0, NCH // 2, body, 0)
        if NCH % 2:
            off = base + (NCH - 1) * _CHUNK
            pltpu.sync_copy(src_hbm.at[pl.ds(off, _CHUNK)], isrc)
            pltpu.sync_copy(dst_hbm.at[pl.ds(off, _CHUNK)], idst)
            pltpu.async_copy(x_hbm.at[isrc], rows, sem).wait()
            pltpu.sync_copy(rows, acc.at[idst], add=True)

        plsc.subcore_barrier()
        _drain_spmem(acc, rows, sums_out.at[c], s, RPT, TAIL0, TAILN)

    return k


def _make_agg2(N, Dh, E):
    EPT = E // _NS
    NCH = EPT // _CHUNK
    RPT, TAIL0, TAILN = _spmem_rows(N)
    mesh = plsc.VectorSubcoreMesh(
        core_axis_name="c", subcore_axis_name="s",
        num_cores=_NC_SC, num_subcores=_NS)

    @functools.partial(
        pl.kernel,
        out_type=jax.ShapeDtypeStruct((_NC_SC, N, Dh), jnp.float32),
        mesh=mesh,
        scratch_types=[
            pltpu.VMEM((_CHUNK,), jnp.int32),
            pltpu.VMEM((_CHUNK,), jnp.int32),
            pltpu.VMEM((_CHUNK,), jnp.int32),
            pltpu.VMEM((_CHUNK,), jnp.int32),
            pltpu.VMEM((_CHUNK, Dh), jnp.float32),
            pltpu.VMEM((_CHUNK, Dh), jnp.float32),
            pltpu.VMEM((_CHUNK, Dh), jnp.float32),
            pltpu.VMEM_SHARED((N, Dh), jnp.float32),
            pltpu.SemaphoreType.DMA,
            pltpu.SemaphoreType.DMA,
        ],
    )
    def k(tab_hbm, src_hbm, dst_hbm, sums_out,
          isrc, idst, isrc2, idst2, rows, rows2, stg, acc, sem, sem2):
        c = lax.axis_index("c")
        s = lax.axis_index("s")

        _fill_const(stg, _CHUNK, Dh, 0.0)
        _zero_spmem(acc, stg, s, RPT, TAIL0, TAILN)
        plsc.subcore_barrier()

        base = s * EPT
        shift = c * N

        def body(k2, _):
            off0 = base + (2 * k2) * _CHUNK
            off1 = off0 + _CHUNK
            pltpu.sync_copy(src_hbm.at[pl.ds(off0, _CHUNK)], isrc)
            pltpu.sync_copy(dst_hbm.at[pl.ds(off0, _CHUNK)], idst)
            for j in range(_CHUNK // 16):
                sl = pl.ds(j * 16, 16)
                isrc[sl] = isrc[sl] + shift
            cp0 = pltpu.async_copy(tab_hbm.at[isrc], rows, sem)
            pltpu.sync_copy(src_hbm.at[pl.ds(off1, _CHUNK)], isrc2)
            pltpu.sync_copy(dst_hbm.at[pl.ds(off1, _CHUNK)], idst2)
            for j in range(_CHUNK // 16):
                sl = pl.ds(j * 16, 16)
                isrc2[sl] = isrc2[sl] + shift
            cp1 = pltpu.async_copy(tab_hbm.at[isrc2], rows2, sem2)
            cp0.wait()
            pltpu.sync_copy(rows, acc.at[idst], add=True)
            cp1.wait()
            pltpu.sync_copy(rows2, acc.at[idst2], add=True)
            return 0
        lax.fori_loop(0, NCH // 2, body, 0)
        if NCH % 2:
            off = base + (NCH - 1) * _CHUNK
            pltpu.sync_copy(src_hbm.at[pl.ds(off, _CHUNK)], isrc)
            pltpu.sync_copy(dst_hbm.at[pl.ds(off, _CHUNK)], idst)
            for j in range(_CHUNK // 16):
                sl = pl.ds(j * 16, 16)
                isrc[sl] = isrc[sl] + shift
            pltpu.async_copy(tab_hbm.at[isrc], rows, sem).wait()
            pltpu.sync_copy(rows, acc.at[idst], add=True)

        plsc.subcore_barrier()
        _drain_spmem(acc, stg, sums_out.at[c], s, RPT, TAIL0, TAILN)

    return k


def _stageB_body(sums_ref, cnts_ref, x_ref, w1l_ref, b1_ref, w1r_ref,
                 w2l_ref, w2r_ref, p2_ref, r2_ref, *, half):
    ssum = sums_ref[0] + sums_ref[1]
    cnt = cnts_ref[0, :, 0:1] + cnts_ref[1, :, 0:1]
    inv = 1.0 / jnp.maximum(cnt, 1.0)
    mean = ssum * inv
    h1 = jnp.maximum(
        jnp.dot(mean, w1l_ref[...], preferred_element_type=jnp.float32)
        + jnp.dot(x_ref[...], w1r_ref[...], preferred_element_type=jnp.float32)
        + b1_ref[...], 0.0)
    p2 = jnp.dot(h1, w2l_ref[...], preferred_element_type=jnp.float32)
    p2_ref[0] = p2[:, :half]
    p2_ref[1] = p2[:, half:]
    r2_ref[...] = jnp.dot(h1, w2r_ref[...], preferred_element_type=jnp.float32)


def _stageC_body(sums2_ref, cnts_ref, r2_ref, b2_ref, wh0_ref, bh0_ref,
                 wh1_ref, bh1_ref, wh2_ref, bh2_ref, wf_ref, bf_ref,
                 out_ref, *, n_cls):
    cnt = cnts_ref[0, :, 0:1] + cnts_ref[1, :, 0:1]
    inv = 1.0 / jnp.maximum(cnt, 1.0)
    mean2 = jnp.concatenate([sums2_ref[0], sums2_ref[1]], axis=1) * inv
    h = jnp.maximum(mean2 + b2_ref[...] + r2_ref[...], 0.0)
    h = jnp.maximum(jnp.dot(h, wh0_ref[...], preferred_element_type=jnp.float32) + bh0_ref[...], 0.0)
    h = jnp.maximum(jnp.dot(h, wh1_ref[...], preferred_element_type=jnp.float32) + bh1_ref[...], 0.0)
    h = jnp.maximum(jnp.dot(h, wh2_ref[...], preferred_element_type=jnp.float32) + bh2_ref[...], 0.0)
    logits = jnp.dot(h, wf_ref[...], preferred_element_type=jnp.float32) + bf_ref[...]
    col = lax.broadcasted_iota(jnp.int32, logits.shape, 1)
    masked = jnp.where(col < n_cls, logits, -1e30)
    m = jnp.max(masked, axis=1, keepdims=True)
    e = jnp.exp(masked - m)
    probs = e / jnp.sum(e, axis=1, keepdims=True)
    out_ref[...] = probs[:, :n_cls]


def kernel(x, edge_index, W1l, b1, W1r, W2l, b2, W2r,
           Wh0, bh0, Wh1, bh1, Wh2, bh2, Wf, bf):
    N, D = x.shape
    E = edge_index.shape[1]
    H1 = W1l.shape[1]
    H2 = W2l.shape[1]
    NCLS = Wf.shape[1]
    half = H2 // 2
    BN = 400
    grid = (N // BN,)

    src = edge_index[0].astype(jnp.int32)
    dst = edge_index[1].astype(jnp.int32)

    sums1, cnts1 = _make_agg1(N, D, E)(x, src, dst)

    wcopy = lambda shape: pl.BlockSpec(shape, lambda i: tuple(0 for _ in shape))
    b1r = b1.reshape(1, H1)
    stageB = pl.pallas_call(
        functools.partial(_stageB_body, half=half),
        grid=grid,
        in_specs=[
            pl.BlockSpec((_NC_SC, BN, D), lambda i: (0, i, 0)),
            pl.BlockSpec((_NC_SC, BN, D), lambda i: (0, i, 0)),
            pl.BlockSpec((BN, D), lambda i: (i, 0)),
            wcopy((D, H1)), wcopy((1, H1)), wcopy((D, H1)),
            wcopy((H1, H2)), wcopy((H1, H2)),
        ],
        out_specs=[
            pl.BlockSpec((_NC_SC, BN, half), lambda i: (0, i, 0)),
            pl.BlockSpec((BN, H2), lambda i: (i, 0)),
        ],
        out_shape=[
            jax.ShapeDtypeStruct((_NC_SC, N, half), jnp.float32),
            jax.ShapeDtypeStruct((N, H2), jnp.float32),
        ],
    )
    p2, r2 = stageB(sums1, cnts1, x, W1l, b1r, W1r, W2l, W2r)

    sums2 = _make_agg2(N, half, E)(p2.reshape(_NC_SC * N, half), src, dst)

    wf_p = jnp.zeros((H2, 128), jnp.float32).at[:, :NCLS].set(Wf)
    bf_p = jnp.zeros((1, 128), jnp.float32).at[0, :NCLS].set(bf)
    stageC = pl.pallas_call(
        functools.partial(_stageC_body, n_cls=NCLS),
        grid=grid,
        in_specs=[
            pl.BlockSpec((_NC_SC, BN, half), lambda i: (0, i, 0)),
            pl.BlockSpec((_NC_SC, BN, 128), lambda i: (0, i, 0)),
            pl.BlockSpec((BN, H2), lambda i: (i, 0)),
            wcopy((1, H2)),
            wcopy((H2, H2)), wcopy((1, H2)),
            wcopy((H2, H2)), wcopy((1, H2)),
            wcopy((H2, H2)), wcopy((1, H2)),
            wcopy((H2, 128)), wcopy((1, 128)),
        ],
        out_specs=pl.BlockSpec((BN, NCLS), lambda i: (i, 0)),
        out_shape=jax.ShapeDtypeStruct((N, NCLS), jnp.float32),
    )
    return stageC(sums2, cnts1, r2, b2.reshape(1, H2),
                  Wh0, bh0.reshape(1, H2), Wh1, bh1.reshape(1, H2),
                  Wh2, bh2.reshape(1, H2), wf_p, bf_p)
